# baseline (speedup 1.0000x reference)
import jax
import jax.numpy as jnp
from jax import lax
from jax.experimental import pallas as pl
from jax.experimental.pallas import tpu as pltpu
from jax.experimental.pallas import tpu_sc as plsc

N_NODES = 10000
N_EDGES = 320000
D = 128

NC = 2
NS = 16
NW = NC * NS
CH = 128
SB = 4
NSB = 20
NCH = SB * NSB
E_TILE = NCH * CH
E_PAD = NW * E_TILE

ROWS_MAIN = 624
ROWS_TAIL = N_NODES - NS * ROWS_MAIN


def _spmm_body(x_hbm, srcs_hbm, dsts_hbm, norms_hbm, out_hbm,
               src_r, dst_r, norm_r, rows_v, acc_sh, gsem0, gsem1,
               isem0, isem1):
    c = lax.axis_index("c")
    s = lax.axis_index("s")
    wid = s * NC + c
    gsem = (gsem0, gsem1)
    isem = (isem0, isem1)

    zeros16 = jnp.zeros((16,), jnp.float32)

    def _zero_row(r, _):
        for b in range(D // 16):
            rows_v[0, r, pl.ds(b * 16, 16)] = zeros16
        return 0

    lax.fori_loop(0, CH, _zero_row, 0)

    base = s * ROWS_MAIN
    for off in range(0, ROWS_MAIN, CH):
        size = min(CH, ROWS_MAIN - off)
        pltpu.sync_copy(rows_v.at[0, pl.ds(0, size)],
                        acc_sh.at[pl.ds(base + off, size)])

    @pl.when(s == NS - 1)
    def _():
        pltpu.sync_copy(rows_v.at[0, pl.ds(0, ROWS_TAIL)],
                        acc_sh.at[pl.ds(NS * ROWS_MAIN, ROWS_TAIL)])

    plsc.subcore_barrier()

    def _start_idx(sb, r):
        pltpu.async_copy(srcs_hbm.at[wid, sb], src_r.at[r], isem[r])
        pltpu.async_copy(dsts_hbm.at[wid, sb], dst_r.at[r], isem[r])
        pltpu.async_copy(norms_hbm.at[wid, sb], norm_r.at[r], isem[r])

    def _wait_idx(sb, r):
        pltpu.make_async_copy(srcs_hbm.at[wid, sb], src_r.at[r],
                              isem[r]).wait()
        pltpu.make_async_copy(dsts_hbm.at[wid, sb], dst_r.at[r],
                              isem[r]).wait()
        pltpu.make_async_copy(norms_hbm.at[wid, sb], norm_r.at[r],
                              isem[r]).wait()

    def _start_gather(r, jj, slot):
        pltpu.async_copy(x_hbm.at[src_r.at[r, jj]], rows_v.at[slot],
                         gsem[slot])

    def _wait_gather(r, jj, slot):
        pltpu.make_async_copy(x_hbm.at[src_r.at[r, jj]], rows_v.at[slot],
                              gsem[slot]).wait()

    def _scale(r, jj, slot):
        def _scale_grp(g, _):
            nv16 = norm_r[r, jj, pl.ds(g * 16, 16)]
            e0 = g * 16
            for ei in range(16):
                nv = jnp.full((16,), nv16[ei], jnp.float32)
                for b in range(D // 16):
                    sl = pl.ds(b * 16, 16)
                    rows_v[slot, e0 + ei, sl] = rows_v[slot, e0 + ei, sl] * nv
            return 0

        lax.fori_loop(0, CH // 16, _scale_grp, 0)

    def _superblock(sb, r):
        for jj in range(SB):
            j = sb * SB + jj
            slot = jj % 2
            _wait_gather(r, jj, slot)
            _scale(r, jj, slot)
            pltpu.sync_copy(rows_v.at[slot], acc_sh.at[dst_r.at[r, jj]],
                            add=True)

            if jj < SB - 2:
                @pl.when(j < NCH - 2)
                def _():
                    _start_gather(r, jj + 2, slot)
            elif jj == SB - 2:
                @pl.when(j < NCH - 2)
                def _():
                    _wait_idx(sb + 1, 1 - r)
                    _start_gather(1 - r, 0, slot)
            else:
                @pl.when(j < NCH - 2)
                def _():
                    _start_gather(1 - r, 1, slot)

        @pl.when(sb + 2 < NSB)
        def _():
            _start_idx(sb + 2, r)

    _start_idx(0, 0)
    _start_idx(1, 1)
    _wait_idx(0, 0)
    _start_gather(0, 0, 0)
    _start_gather(0, 1, 1)

    def _outer(t, _):
        _superblock(2 * t, 0)
        _superblock(2 * t + 1, 1)
        return 0

    lax.fori_loop(0, NSB // 2, _outer, 0)

    plsc.subcore_barrier()

    pltpu.sync_copy(acc_sh.at[pl.ds(base, ROWS_MAIN)],
                    out_hbm.at[c, pl.ds(base, ROWS_MAIN)])

    @pl.when(s == NS - 1)
    def _():
        pltpu.sync_copy(acc_sh.at[pl.ds(NS * ROWS_MAIN, ROWS_TAIL)],
                        out_hbm.at[c, pl.ds(NS * ROWS_MAIN, ROWS_TAIL)])


_spmm = pl.kernel(
    _spmm_body,
    out_type=jax.ShapeDtypeStruct((NC, N_NODES, D), jnp.float32),
    mesh=plsc.VectorSubcoreMesh(core_axis_name="c", subcore_axis_name="s"),
    scratch_types=[
        pltpu.VMEM((2, SB, CH), jnp.int32),
        pltpu.VMEM((2, SB, CH), jnp.int32),
        pltpu.VMEM((2, SB, CH), jnp.float32),
        pltpu.VMEM((2, CH, D), jnp.float32),
        pltpu.VMEM_SHARED((N_NODES, D), jnp.float32),
        pltpu.SemaphoreType.DMA,
        pltpu.SemaphoreType.DMA,
        pltpu.SemaphoreType.DMA,
        pltpu.SemaphoreType.DMA,
    ],
)


_BLK = 1000


def _mm_body(x_ref, w_ref, o_ref):
    o_ref[...] = jnp.dot(x_ref[...], w_ref[...],
                         preferred_element_type=jnp.float32)


def _mm(x, w):
    n = x.shape[0]
    return pl.pallas_call(
        _mm_body,
        grid=(n // _BLK,),
        in_specs=[pl.BlockSpec((_BLK, D), lambda i: (i, 0)),
                  pl.BlockSpec((D, D), lambda i: (0, 0))],
        out_specs=pl.BlockSpec((_BLK, D), lambda i: (i, 0)),
        out_shape=jax.ShapeDtypeStruct((n, D), jnp.float32),
    )(x, w)


def _fuse_body(p_ref, b_ref, w_ref, o_ref):
    h = p_ref[0] + p_ref[1] + b_ref[...]
    h = jnp.maximum(h, 0.0)
    o_ref[...] = jnp.dot(h, w_ref[...], preferred_element_type=jnp.float32)


def _fuse_relu_mm(parts, b, w):
    n = parts.shape[1]
    return pl.pallas_call(
        _fuse_body,
        grid=(n // _BLK,),
        in_specs=[pl.BlockSpec((2, _BLK, D), lambda i: (0, i, 0)),
                  pl.BlockSpec((1, D), lambda i: (0, 0)),
                  pl.BlockSpec((D, D), lambda i: (0, 0))],
        out_specs=pl.BlockSpec((_BLK, D), lambda i: (i, 0)),
        out_shape=jax.ShapeDtypeStruct((n, D), jnp.float32),
    )(parts, b.reshape(1, D), w)


def _final_body(q_ref, b_ref, o_ref):
    o_ref[...] = q_ref[0] + q_ref[1] + b_ref[...]


def _final_add(parts, b):
    n = parts.shape[1]
    return pl.pallas_call(
        _final_body,
        grid=(n // _BLK,),
        in_specs=[pl.BlockSpec((2, _BLK, D), lambda i: (0, i, 0)),
                  pl.BlockSpec((1, D), lambda i: (0, 0))],
        out_specs=pl.BlockSpec((_BLK, D), lambda i: (i, 0)),
        out_shape=jax.ShapeDtypeStruct((n, D), jnp.float32),
    )(parts, b.reshape(1, D))


def kernel(feat, edge_index, norm_data, W1, b1, W2, b2):
    src = edge_index[0].astype(jnp.int32)
    dst = edge_index[1].astype(jnp.int32)
    norm = norm_data.astype(jnp.float32)

    pad = E_PAD - N_EDGES
    srcs = jnp.concatenate([src, jnp.zeros((pad,), jnp.int32)]).reshape(NW, NSB, SB, CH)
    dsts = jnp.concatenate([dst, jnp.zeros((pad,), jnp.int32)]).reshape(NW, NSB, SB, CH)
    norms = jnp.concatenate([norm, jnp.zeros((pad,), jnp.float32)]).reshape(NW, NSB, SB, CH)

    x1 = _mm(feat, W1)
    p = _spmm(x1, srcs, dsts, norms)
    x2 = _fuse_relu_mm(p, b1, W2)
    q = _spmm(x2, srcs, dsts, norms)
    return _final_add(q, b2)
